# Initial kernel scaffold; baseline (speedup 1.0000x reference)
import functools

import jax
import jax.numpy as jnp
from jax import lax
from jax.experimental import pallas as pl
from jax.experimental.pallas import tpu as pltpu
from jax.experimental.pallas import tpu_sc as plsc

N_NODES = 10000
N_EDGES = 320000
NUM_RELS = 8
DIM = 128

NC = 2
NS = 16
L = 16
NW = NC * NS
EPW = N_EDGES // NW
CHUNK = 128
FULL_CHUNKS = EPW // CHUNK
TAIL = EPW - FULL_CHUNKS * CHUNK
ROWS_PER_TILE = N_NODES // NS


def _mesh():
    return plsc.VectorSubcoreMesh(core_axis_name="c", subcore_axis_name="s")


def _worker_id():
    return lax.axis_index("s") * NC + lax.axis_index("c")


def _iota16():
    return lax.iota(jnp.int32, L)


def _build_counts_kernel():
    out_t = (jax.ShapeDtypeStruct((N_NODES, L), jnp.float32),
             jax.ShapeDtypeStruct((N_NODES, L), jnp.float32))

    @functools.partial(
        pl.kernel,
        out_type=out_t,
        mesh=_mesh(),
        scratch_types=[
            pltpu.VMEM_SHARED((N_NODES, L), jnp.float32),
            pltpu.VMEM((CHUNK,), jnp.int32),
            pltpu.VMEM((CHUNK,), jnp.int32),
            pltpu.VMEM((CHUNK, L), jnp.float32),
        ],
    )
    def counts_kernel(dst_hbm, edt_hbm, out0, out1, cnt_sh, dst_v, edt_v, rows_v):
        c = lax.axis_index("c")
        s = lax.axis_index("s")
        wid = _worker_id()
        zeros = jnp.zeros((L,), jnp.float32)
        ones = jnp.full((L,), 1.0, jnp.float32)

        def zr(i, _):
            rows_v[i] = zeros
            return 0
        lax.fori_loop(0, CHUNK, zr, 0)

        nz = ROWS_PER_TILE
        j = 0
        while nz > 0:
            n = min(CHUNK, nz)
            pltpu.sync_copy(rows_v.at[pl.ds(0, n)],
                            cnt_sh.at[pl.ds(s * ROWS_PER_TILE + j, n)])
            nz -= n
            j += n
        plsc.subcore_barrier()

        def do_chunk(base, n):
            base = pl.multiple_of(base, 8)
            pltpu.sync_copy(dst_hbm.at[pl.ds(base, n)], dst_v.at[pl.ds(0, n)])
            pltpu.sync_copy(edt_hbm.at[pl.ds(base, n)], edt_v.at[pl.ds(0, n)])
            for g in range(n // L):
                ridx = jnp.full((L,), g * L, jnp.int32) + _iota16()
                col = edt_v[pl.ds(g * L, L)]
                plsc.store_scatter(rows_v, [ridx, col], ones)
            pltpu.sync_copy(rows_v.at[pl.ds(0, n)],
                            cnt_sh.at[dst_v.at[pl.ds(0, n)]], add=True)
            for g in range(n // L):
                ridx = jnp.full((L,), g * L, jnp.int32) + _iota16()
                col = edt_v[pl.ds(g * L, L)]
                plsc.store_scatter(rows_v, [ridx, col], zeros)

        def body(k, _):
            do_chunk(wid * EPW + k * CHUNK, CHUNK)
            return 0
        lax.fori_loop(0, FULL_CHUNKS, body, 0)
        if TAIL:
            do_chunk(wid * EPW + FULL_CHUNKS * CHUNK, TAIL)

        plsc.subcore_barrier()
        row0 = s * ROWS_PER_TILE

        @pl.when(c == 0)
        def _():
            pltpu.sync_copy(cnt_sh.at[pl.ds(row0, ROWS_PER_TILE)],
                            out0.at[pl.ds(row0, ROWS_PER_TILE)])

        @pl.when(c == 1)
        def _():
            pltpu.sync_copy(cnt_sh.at[pl.ds(row0, ROWS_PER_TILE)],
                            out1.at[pl.ds(row0, ROWS_PER_TILE)])

    return counts_kernel


def _build_norm_kernel():
    @functools.partial(
        pl.kernel,
        out_type=jax.ShapeDtypeStruct((N_EDGES,), jnp.float32),
        mesh=_mesh(),
        scratch_types=[
            pltpu.VMEM((CHUNK,), jnp.int32),
            pltpu.VMEM((CHUNK,), jnp.int32),
            pltpu.VMEM((CHUNK, L), jnp.float32),
            pltpu.VMEM((CHUNK, L), jnp.float32),
            pltpu.VMEM((CHUNK,), jnp.float32),
        ],
    )
    def norm_kernel(cnt0, cnt1, dst_hbm, edt_hbm, en_hbm,
                    dst_v, edt_v, rows0, rows1, en_v):
        wid = _worker_id()

        def do_chunk(base, n):
            base = pl.multiple_of(base, 8)
            pltpu.sync_copy(dst_hbm.at[pl.ds(base, n)], dst_v.at[pl.ds(0, n)])
            pltpu.sync_copy(edt_hbm.at[pl.ds(base, n)], edt_v.at[pl.ds(0, n)])
            pltpu.sync_copy(cnt0.at[dst_v.at[pl.ds(0, n)]], rows0.at[pl.ds(0, n)])
            pltpu.sync_copy(cnt1.at[dst_v.at[pl.ds(0, n)]], rows1.at[pl.ds(0, n)])
            for g in range(n // L):
                ridx = jnp.full((L,), g * L, jnp.int32) + _iota16()
                col = edt_v[pl.ds(g * L, L)]
                c0 = plsc.load_gather(rows0, [ridx, col])
                c1 = plsc.load_gather(rows1, [ridx, col])
                cnt = c0 + c1
                en_v[pl.ds(g * L, L)] = 1.0 / jnp.maximum(cnt, 1.0)
            pltpu.sync_copy(en_v.at[pl.ds(0, n)], en_hbm.at[pl.ds(base, n)])

        def body(k, _):
            do_chunk(wid * EPW + k * CHUNK, CHUNK)
            return 0
        lax.fori_loop(0, FULL_CHUNKS, body, 0)
        if TAIL:
            do_chunk(wid * EPW + FULL_CHUNKS * CHUNK, TAIL)

    return norm_kernel


def _build_layer_kernel():
    out_t = (jax.ShapeDtypeStruct((N_NODES, DIM), jnp.float32),
             jax.ShapeDtypeStruct((N_NODES, DIM), jnp.float32))

    @functools.partial(
        pl.kernel,
        out_type=out_t,
        mesh=_mesh(),
        scratch_types=[
            pltpu.VMEM_SHARED((N_NODES, DIM), jnp.float32),
            pltpu.VMEM((CHUNK,), jnp.int32),
            pltpu.VMEM((CHUNK,), jnp.int32),
            pltpu.VMEM((CHUNK,), jnp.int32),
            pltpu.VMEM((CHUNK,), jnp.int32),
            pltpu.VMEM((CHUNK,), jnp.float32),
            pltpu.VMEM((CHUNK, DIM), jnp.float32),
        ],
    )
    def layer_kernel(xw_hbm, src_hbm, edt_hbm, dst_hbm, en_hbm, out0, out1,
                     agg_sh, src_v, edt_v, dst_v, gidx_v, en_v, rows_v):
        c = lax.axis_index("c")
        s = lax.axis_index("s")
        wid = _worker_id()
        zeros = jnp.zeros((L,), jnp.float32)

        def zr(i, _):
            for k in range(DIM // L):
                rows_v[i, pl.ds(k * L, L)] = zeros
            return 0
        lax.fori_loop(0, CHUNK, zr, 0)
        nz = ROWS_PER_TILE
        j = 0
        while nz > 0:
            n = min(CHUNK, nz)
            pltpu.sync_copy(rows_v.at[pl.ds(0, n)],
                            agg_sh.at[pl.ds(s * ROWS_PER_TILE + j, n)])
            nz -= n
            j += n
        plsc.subcore_barrier()

        def do_chunk(base, n):
            base = pl.multiple_of(base, 8)
            pltpu.sync_copy(src_hbm.at[pl.ds(base, n)], src_v.at[pl.ds(0, n)])
            pltpu.sync_copy(edt_hbm.at[pl.ds(base, n)], edt_v.at[pl.ds(0, n)])
            pltpu.sync_copy(dst_hbm.at[pl.ds(base, n)], dst_v.at[pl.ds(0, n)])
            pltpu.sync_copy(en_hbm.at[pl.ds(base, n)], en_v.at[pl.ds(0, n)])
            for g in range(n // L):
                sv = src_v[pl.ds(g * L, L)]
                ev = edt_v[pl.ds(g * L, L)]
                gidx_v[pl.ds(g * L, L)] = ev * N_NODES + sv
            pltpu.sync_copy(xw_hbm.at[gidx_v.at[pl.ds(0, n)]],
                            rows_v.at[pl.ds(0, n)])

            def scale(i, _):
                f = lax.broadcast(en_v[i], (L,))
                for k in range(DIM // L):
                    rows_v[i, pl.ds(k * L, L)] = rows_v[i, pl.ds(k * L, L)] * f
                return 0
            lax.fori_loop(0, n, scale, 0)

            pltpu.sync_copy(rows_v.at[pl.ds(0, n)],
                            agg_sh.at[dst_v.at[pl.ds(0, n)]], add=True)

        def body(k, _):
            do_chunk(wid * EPW + k * CHUNK, CHUNK)
            return 0
        lax.fori_loop(0, FULL_CHUNKS, body, 0)
        if TAIL:
            do_chunk(wid * EPW + FULL_CHUNKS * CHUNK, TAIL)

        plsc.subcore_barrier()
        row0 = s * ROWS_PER_TILE

        @pl.when(c == 0)
        def _():
            pltpu.sync_copy(agg_sh.at[pl.ds(row0, ROWS_PER_TILE)],
                            out0.at[pl.ds(row0, ROWS_PER_TILE)])

        @pl.when(c == 1)
        def _():
            pltpu.sync_copy(agg_sh.at[pl.ds(row0, ROWS_PER_TILE)],
                            out1.at[pl.ds(row0, ROWS_PER_TILE)])

    return layer_kernel


_MB = 1000
_NB = N_NODES // _MB


def _einsum_tc(x, W):
    def body(x_ref, w_ref, o_ref):
        o_ref[0] = jnp.dot(x_ref[...], w_ref[0],
                           preferred_element_type=jnp.float32)

    return pl.pallas_call(
        body,
        grid=(_NB, NUM_RELS),
        in_specs=[
            pl.BlockSpec((_MB, DIM), lambda b, r: (b, 0)),
            pl.BlockSpec((1, DIM, DIM), lambda b, r: (r, 0, 0)),
        ],
        out_specs=pl.BlockSpec((1, _MB, DIM), lambda b, r: (r, b, 0)),
        out_shape=jax.ShapeDtypeStruct((NUM_RELS, N_NODES, DIM), jnp.float32),
    )(x, W)


def _combine_tc(p0, p1, x, root, b, relu):
    def body(p0_ref, p1_ref, x_ref, r_ref, b_ref, o_ref):
        acc = p0_ref[...] + p1_ref[...] + jnp.dot(
            x_ref[...], r_ref[...], preferred_element_type=jnp.float32)
        acc = acc + b_ref[...]
        if relu:
            acc = jnp.maximum(acc, 0.0)
        o_ref[...] = acc

    return pl.pallas_call(
        body,
        grid=(_NB,),
        in_specs=[
            pl.BlockSpec((_MB, DIM), lambda b: (b, 0)),
            pl.BlockSpec((_MB, DIM), lambda b: (b, 0)),
            pl.BlockSpec((_MB, DIM), lambda b: (b, 0)),
            pl.BlockSpec((DIM, DIM), lambda b: (0, 0)),
            pl.BlockSpec((1, DIM), lambda b: (0, 0)),
        ],
        out_specs=pl.BlockSpec((_MB, DIM), lambda b: (b, 0)),
        out_shape=jax.ShapeDtypeStruct((N_NODES, DIM), jnp.float32),
    )(p0, p1, x, root, b)


def kernel(x, edge_index, edge_type, W1, root1, b1, W2, root2, b2):
    src = edge_index[0].astype(jnp.int32)
    dst = edge_index[1].astype(jnp.int32)
    edt = edge_type.astype(jnp.int32)
    b1r = b1.reshape(1, DIM)
    b2r = b2.reshape(1, DIM)

    counts_k = _build_counts_kernel()
    norm_k = _build_norm_kernel()
    layer_k = _build_layer_kernel()

    cnt0, cnt1 = counts_k(dst, edt)
    en = norm_k(cnt0, cnt1, dst, edt)

    xw1 = _einsum_tc(x, W1).reshape(NUM_RELS * N_NODES, DIM)
    a0, a1 = layer_k(xw1, src, edt, dst, en)
    h = _combine_tc(a0, a1, x, root1, b1r, relu=True)

    xw2 = _einsum_tc(h, W2).reshape(NUM_RELS * N_NODES, DIM)
    c0, c1 = layer_k(xw2, src, edt, dst, en)
    out = _combine_tc(c0, c1, h, root2, b2r, relu=False)
    return out


# speedup vs baseline: 13.4403x; 13.4403x over previous
import functools

import jax
import jax.numpy as jnp
from jax import lax
from jax.experimental import pallas as pl
from jax.experimental.pallas import tpu as pltpu
from jax.experimental.pallas import tpu_sc as plsc

N_NODES = 10000
N_PAD = 10240
N_EDGES = 320000
NUM_RELS = 8
DIM = 128
NKEYS = N_NODES * NUM_RELS

NC = 2
NS = 16
L = 16
NW = NC * NS
EPW = N_EDGES // NW
CHUNK = 128
FULL_CHUNKS = EPW // CHUNK
TAIL = EPW - FULL_CHUNKS * CHUNK
KEYS_PER_TILE = NKEYS // NS
AGG_ROWS_PER_TILE = N_PAD // NS


def _mesh():
    return plsc.VectorSubcoreMesh(core_axis_name="c", subcore_axis_name="s")


def _worker_id():
    return lax.axis_index("s") * NC + lax.axis_index("c")


def _build_counts_kernel():
    out_t = (jax.ShapeDtypeStruct((NKEYS,), jnp.float32),
             jax.ShapeDtypeStruct((NKEYS,), jnp.float32))

    @functools.partial(
        pl.kernel,
        out_type=out_t,
        mesh=_mesh(),
        scratch_types=[
            pltpu.VMEM_SHARED((NKEYS,), jnp.float32),
            pltpu.VMEM((CHUNK,), jnp.int32),
            pltpu.VMEM((CHUNK,), jnp.int32),
            pltpu.VMEM((CHUNK,), jnp.int32),
            pltpu.VMEM((CHUNK,), jnp.float32),
            pltpu.VMEM((1024,), jnp.float32),
        ],
    )
    def counts_kernel(dst_hbm, edt_hbm, out0, out1,
                      cnt_sh, dst_v, edt_v, key_v, ones_v, zbuf):
        c = lax.axis_index("c")
        s = lax.axis_index("s")
        wid = _worker_id()
        ones = jnp.full((L,), 1.0, jnp.float32)
        zeros = jnp.zeros((L,), jnp.float32)
        for g in range(CHUNK // L):
            ones_v[pl.ds(g * L, L)] = ones

        def zb(i, _):
            zbuf[pl.ds(i * L, L)] = zeros
            return 0
        lax.fori_loop(0, 1024 // L, zb, 0)

        base0 = s * KEYS_PER_TILE
        for j, n in ((0, 1024), (1024, 1024), (2048, 1024), (3072, 1024),
                     (4096, 904)):
            pltpu.sync_copy(zbuf.at[pl.ds(0, n)],
                            cnt_sh.at[pl.ds(base0 + j, n)])
        plsc.subcore_barrier()

        def do_chunk(base, n):
            base = pl.multiple_of(base, 8)
            pltpu.sync_copy(dst_hbm.at[pl.ds(base, n)], dst_v.at[pl.ds(0, n)])
            pltpu.sync_copy(edt_hbm.at[pl.ds(base, n)], edt_v.at[pl.ds(0, n)])
            for g in range(n // L):
                dv = dst_v[pl.ds(g * L, L)]
                ev = edt_v[pl.ds(g * L, L)]
                key_v[pl.ds(g * L, L)] = dv * NUM_RELS + ev
            pltpu.sync_copy(ones_v.at[pl.ds(0, n)],
                            cnt_sh.at[key_v.at[pl.ds(0, n)]], add=True)

        def body(k, _):
            do_chunk(wid * EPW + k * CHUNK, CHUNK)
            return 0
        lax.fori_loop(0, FULL_CHUNKS, body, 0)
        if TAIL:
            do_chunk(wid * EPW + FULL_CHUNKS * CHUNK, TAIL)

        plsc.subcore_barrier()

        def wb(out):
            for j, n in ((0, 1024), (1024, 1024), (2048, 1024), (3072, 1024),
                         (4096, 904)):
                pltpu.sync_copy(cnt_sh.at[pl.ds(base0 + j, n)],
                                zbuf.at[pl.ds(0, n)])
                pltpu.sync_copy(zbuf.at[pl.ds(0, n)],
                                out.at[pl.ds(base0 + j, n)])

        @pl.when(c == 0)
        def _():
            wb(out0)

        @pl.when(c == 1)
        def _():
            wb(out1)

    return counts_kernel


def _build_norm_kernel():
    @functools.partial(
        pl.kernel,
        out_type=jax.ShapeDtypeStruct((N_EDGES,), jnp.float32),
        mesh=_mesh(),
        scratch_types=[
            pltpu.VMEM((CHUNK,), jnp.int32),
            pltpu.VMEM((CHUNK,), jnp.int32),
            pltpu.VMEM((CHUNK,), jnp.int32),
            pltpu.VMEM((CHUNK,), jnp.float32),
            pltpu.VMEM((CHUNK,), jnp.float32),
            pltpu.VMEM((CHUNK,), jnp.float32),
        ],
    )
    def norm_kernel(cnt0, cnt1, dst_hbm, edt_hbm, en_hbm,
                    dst_v, edt_v, key_v, c0_v, c1_v, en_v):
        wid = _worker_id()

        def do_chunk(base, n):
            base = pl.multiple_of(base, 8)
            pltpu.sync_copy(dst_hbm.at[pl.ds(base, n)], dst_v.at[pl.ds(0, n)])
            pltpu.sync_copy(edt_hbm.at[pl.ds(base, n)], edt_v.at[pl.ds(0, n)])
            for g in range(n // L):
                dv = dst_v[pl.ds(g * L, L)]
                ev = edt_v[pl.ds(g * L, L)]
                key_v[pl.ds(g * L, L)] = dv * NUM_RELS + ev
            pltpu.sync_copy(cnt0.at[key_v.at[pl.ds(0, n)]], c0_v.at[pl.ds(0, n)])
            pltpu.sync_copy(cnt1.at[key_v.at[pl.ds(0, n)]], c1_v.at[pl.ds(0, n)])
            for g in range(n // L):
                cnt = c0_v[pl.ds(g * L, L)] + c1_v[pl.ds(g * L, L)]
                en_v[pl.ds(g * L, L)] = 1.0 / jnp.maximum(cnt, 1.0)
            pltpu.sync_copy(en_v.at[pl.ds(0, n)], en_hbm.at[pl.ds(base, n)])

        def body(k, _):
            do_chunk(wid * EPW + k * CHUNK, CHUNK)
            return 0
        lax.fori_loop(0, FULL_CHUNKS, body, 0)
        if TAIL:
            do_chunk(wid * EPW + FULL_CHUNKS * CHUNK, TAIL)

    return norm_kernel


def _build_layer_kernel():
    out_t = (jax.ShapeDtypeStruct((N_PAD, DIM), jnp.float32),
             jax.ShapeDtypeStruct((N_PAD, DIM), jnp.float32))

    @functools.partial(
        pl.kernel,
        out_type=out_t,
        mesh=_mesh(),
        scratch_types=[
            pltpu.VMEM_SHARED((N_PAD, DIM), jnp.float32),
            pltpu.VMEM((CHUNK,), jnp.int32),
            pltpu.VMEM((CHUNK,), jnp.int32),
            pltpu.VMEM((CHUNK,), jnp.int32),
            pltpu.VMEM((CHUNK,), jnp.int32),
            pltpu.VMEM((CHUNK,), jnp.float32),
            pltpu.VMEM((CHUNK, DIM), jnp.float32),
        ],
    )
    def layer_kernel(xw_hbm, src_hbm, edt_hbm, dst_hbm, en_hbm, out0, out1,
                     agg_sh, src_v, edt_v, dst_v, gidx_v, en_v, rows_v):
        c = lax.axis_index("c")
        s = lax.axis_index("s")
        wid = _worker_id()
        zeros = jnp.zeros((L,), jnp.float32)

        def zr(i, _):
            for k in range(DIM // L):
                rows_v[i, pl.ds(k * L, L)] = zeros
            return 0
        lax.fori_loop(0, CHUNK, zr, 0)
        row0 = s * AGG_ROWS_PER_TILE
        for j in range(AGG_ROWS_PER_TILE // CHUNK):
            pltpu.sync_copy(rows_v,
                            agg_sh.at[pl.ds(row0 + j * CHUNK, CHUNK)])
        plsc.subcore_barrier()

        def do_chunk(base, n):
            base = pl.multiple_of(base, 8)
            pltpu.sync_copy(src_hbm.at[pl.ds(base, n)], src_v.at[pl.ds(0, n)])
            pltpu.sync_copy(edt_hbm.at[pl.ds(base, n)], edt_v.at[pl.ds(0, n)])
            pltpu.sync_copy(dst_hbm.at[pl.ds(base, n)], dst_v.at[pl.ds(0, n)])
            pltpu.sync_copy(en_hbm.at[pl.ds(base, n)], en_v.at[pl.ds(0, n)])
            for g in range(n // L):
                sv = src_v[pl.ds(g * L, L)]
                ev = edt_v[pl.ds(g * L, L)]
                gidx_v[pl.ds(g * L, L)] = ev * N_NODES + sv
            pltpu.sync_copy(xw_hbm.at[gidx_v.at[pl.ds(0, n)]],
                            rows_v.at[pl.ds(0, n)])

            def scale(g, _):
                env = en_v[pl.ds(g * L, L)]
                for i in range(L):
                    f = lax.broadcast(env[i], (L,))
                    row = g * L + i
                    for k in range(DIM // L):
                        rows_v[row, pl.ds(k * L, L)] = (
                            rows_v[row, pl.ds(k * L, L)] * f)
                return 0
            lax.fori_loop(0, n // L, scale, 0)

            pltpu.sync_copy(rows_v.at[pl.ds(0, n)],
                            agg_sh.at[dst_v.at[pl.ds(0, n)]], add=True)

        def body(k, _):
            do_chunk(wid * EPW + k * CHUNK, CHUNK)
            return 0
        lax.fori_loop(0, FULL_CHUNKS, body, 0)
        if TAIL:
            do_chunk(wid * EPW + FULL_CHUNKS * CHUNK, TAIL)

        plsc.subcore_barrier()

        def wb(out):
            for j in range(AGG_ROWS_PER_TILE // CHUNK):
                pltpu.sync_copy(agg_sh.at[pl.ds(row0 + j * CHUNK, CHUNK)],
                                rows_v)
                pltpu.sync_copy(rows_v,
                                out.at[pl.ds(row0 + j * CHUNK, CHUNK)])

        @pl.when(c == 0)
        def _():
            wb(out0)

        @pl.when(c == 1)
        def _():
            wb(out1)

    return layer_kernel


_MB = 1000
_NB = N_NODES // _MB


def _einsum_tc(x, W):
    def body(x_ref, w_ref, o_ref):
        o_ref[0] = jnp.dot(x_ref[...], w_ref[0],
                           preferred_element_type=jnp.float32)

    return pl.pallas_call(
        body,
        grid=(_NB, NUM_RELS),
        in_specs=[
            pl.BlockSpec((_MB, DIM), lambda b, r: (b, 0)),
            pl.BlockSpec((1, DIM, DIM), lambda b, r: (r, 0, 0)),
        ],
        out_specs=pl.BlockSpec((1, _MB, DIM), lambda b, r: (r, b, 0)),
        out_shape=jax.ShapeDtypeStruct((NUM_RELS, N_NODES, DIM), jnp.float32),
    )(x, W)


def _combine_tc(p0, p1, x, root, b, relu):
    def body(p0_ref, p1_ref, x_ref, r_ref, b_ref, o_ref):
        acc = p0_ref[...] + p1_ref[...] + jnp.dot(
            x_ref[...], r_ref[...], preferred_element_type=jnp.float32)
        acc = acc + b_ref[...]
        if relu:
            acc = jnp.maximum(acc, 0.0)
        o_ref[...] = acc

    return pl.pallas_call(
        body,
        grid=(_NB,),
        in_specs=[
            pl.BlockSpec((_MB, DIM), lambda b: (b, 0)),
            pl.BlockSpec((_MB, DIM), lambda b: (b, 0)),
            pl.BlockSpec((_MB, DIM), lambda b: (b, 0)),
            pl.BlockSpec((DIM, DIM), lambda b: (0, 0)),
            pl.BlockSpec((1, DIM), lambda b: (0, 0)),
        ],
        out_specs=pl.BlockSpec((_MB, DIM), lambda b: (b, 0)),
        out_shape=jax.ShapeDtypeStruct((N_NODES, DIM), jnp.float32),
    )(p0, p1, x, root, b)


def kernel(x, edge_index, edge_type, W1, root1, b1, W2, root2, b2):
    src = edge_index[0].astype(jnp.int32)
    dst = edge_index[1].astype(jnp.int32)
    edt = edge_type.astype(jnp.int32)
    b1r = b1.reshape(1, DIM)
    b2r = b2.reshape(1, DIM)

    counts_k = _build_counts_kernel()
    norm_k = _build_norm_kernel()
    layer_k = _build_layer_kernel()

    cnt0, cnt1 = counts_k(dst, edt)
    en = norm_k(cnt0, cnt1, dst, edt)

    xw1 = _einsum_tc(x, W1).reshape(NUM_RELS * N_NODES, DIM)
    a0, a1 = layer_k(xw1, src, edt, dst, en)
    h = _combine_tc(a0, a1, x, root1, b1r, relu=True)

    xw2 = _einsum_tc(h, W2).reshape(NUM_RELS * N_NODES, DIM)
    c0, c1 = layer_k(xw2, src, edt, dst, en)
    out = _combine_tc(c0, c1, h, root2, b2r, relu=False)
    return out
